# baseline (speedup 1.0000x reference)
import dataclasses
import functools
import math

import jax
import jax.numpy as jnp
from jax import lax
from jax.experimental import pallas as pl
from jax.experimental.pallas import tpu as pltpu
from jax.experimental.pallas import tpu_sc as plsc

N = 10000
E = 320000
D = 128
T = 16
OUT = 128

NC = 2
NS = 16
WIN = 128
TPW = 80
CH = 8
T0 = 128
T1 = 32
NW0 = NS * T0
EP = NC * NS * TPW * WIN
EROWS = EP // WIN
NP = 10240
RPT = NP // NS
NPS = 10112
RPS = NPS // NS

NB = 25
BN_ = N // NB

_mesh = plsc.VectorSubcoreMesh(core_axis_name="c", subcore_axis_name="s")

_sc_params = pltpu.CompilerParams()
if "needs_layout_passes" in pltpu.CompilerParams.__dataclass_fields__:
    _sc_params = dataclasses.replace(_sc_params, needs_layout_passes=False)


@functools.partial(
    pl.kernel,
    out_type=jax.ShapeDtypeStruct((NC, NP, 16), jnp.float32),
    mesh=_mesh,
    scratch_types=[
        pltpu.VMEM_SHARED((NP, 16), jnp.float32),
        pltpu.VMEM((TPW, WIN), jnp.int32),
        pltpu.VMEM((TPW, WIN), jnp.float32),
        pltpu.VMEM((WIN, 16), jnp.float32),
        pltpu.VMEM((RPT, 16), jnp.float32),
    ],
    compiler_params=_sc_params,
)
def _deg_kernel(col_hbm, ew_hbm, out_hbm, deg_sh, col_v, ew_v, buf, zbuf):
    cid = lax.axis_index("c")
    sid = lax.axis_index("s")
    tile = cid * NS + sid
    zero16 = jnp.zeros((16,), jnp.float32)

    @pl.loop(0, RPT)
    def _(r):
        zbuf[r, :] = zero16

    pltpu.sync_copy(zbuf, deg_sh.at[pl.ds(sid * RPT, RPT)])

    @pl.loop(0, WIN)
    def _(r):
        buf[r, :] = zero16

    base = tile * TPW
    pltpu.sync_copy(col_hbm.at[pl.ds(base, TPW)], col_v)
    pltpu.sync_copy(ew_hbm.at[pl.ds(base, TPW)], ew_v)

    iota16 = lax.iota(jnp.int32, 16)
    zi16 = jnp.zeros((16,), jnp.int32)
    plsc.subcore_barrier()

    @pl.loop(0, TPW)
    def _(j):
        @pl.loop(0, WIN // 16)
        def _(grp):
            ew_reg = ew_v[j, pl.ds(grp * 16, 16)]
            plsc.store_scatter(buf, [iota16 + grp * 16, zi16], ew_reg)

        pltpu.sync_copy(buf, deg_sh.at[col_v.at[j]], add=True)

    plsc.subcore_barrier()
    pltpu.sync_copy(
        deg_sh.at[pl.ds(sid * RPT, RPT)],
        out_hbm.at[cid, pl.ds(sid * RPT, RPT)],
    )


@functools.partial(
    pl.kernel,
    out_type=jax.ShapeDtypeStruct((NC, NPS, OUT), jnp.float32),
    mesh=_mesh,
    scratch_types=[
        pltpu.VMEM_SHARED((NPS, OUT), jnp.float32),
        pltpu.VMEM((2 * CH, WIN), jnp.int32),
        pltpu.VMEM((2 * CH, WIN), jnp.int32),
        pltpu.VMEM((2 * CH, WIN), jnp.float32),
        pltpu.VMEM((WIN, OUT), jnp.float32),
        pltpu.VMEM((WIN, OUT), jnp.float32),
        pltpu.SemaphoreType.DMA,
        pltpu.SemaphoreType.DMA,
        pltpu.SemaphoreType.DMA,
        pltpu.SemaphoreType.DMA,
        pltpu.SemaphoreType.DMA,
        pltpu.SemaphoreType.DMA,
        pltpu.SemaphoreType.DMA,
    ],
    compiler_params=_sc_params,
)
def _prop_kernel(g_hbm, row_hbm, col_hbm, ew_hbm, out_hbm,
                 s_sh, row_st, col_st, ew_st, msA, msB,
                 gsA0, gsA1, gsB0, gsB1, ssA, ssB, stsem):
    cid = lax.axis_index("c")
    sid = lax.axis_index("s")
    zero16 = jnp.zeros((16,), jnp.float32)

    base = jnp.where(cid == 0, sid * T0, NW0 + sid * T1)
    nwin = jnp.where(cid == 0, T0, T1)
    npair = nwin // 2

    @pl.loop(0, WIN)
    def _(r):
        @pl.loop(0, OUT // 16)
        def _(cc):
            msA[r, pl.ds(cc * 16, 16)] = zero16

    @pl.loop(0, RPS // WIN)
    def _(m):
        pltpu.sync_copy(msA, s_sh.at[pl.ds(sid * RPS + m * WIN, WIN)])
    _TAIL = RPS - (RPS // WIN) * WIN
    if _TAIL:
        pltpu.sync_copy(
            msA.at[pl.ds(0, _TAIL)],
            s_sh.at[pl.ds(sid * RPS + (RPS // WIN) * WIN, _TAIL)])

    pltpu.sync_copy(row_hbm.at[pl.ds(base, CH)], row_st.at[pl.ds(0, CH)])
    pltpu.sync_copy(col_hbm.at[pl.ds(base, CH)], col_st.at[pl.ds(0, CH)])
    pltpu.sync_copy(ew_hbm.at[pl.ds(base, CH)], ew_st.at[pl.ds(0, CH)])

    plsc.subcore_barrier()

    def rowof(w):
        return ((w // CH) % 2) * CH + w % CH

    HW = WIN // 2

    def g_issue_h(jj, buf, h, gsem):
        pltpu.async_copy(
            g_hbm.at[row_st.at[jj, pl.ds(h * HW, HW)]],
            buf.at[pl.ds(h * HW, HW)], gsem)

    def g_wait_h(jj, buf, h, gsem):
        pltpu.make_async_copy(
            g_hbm.at[row_st.at[jj, pl.ds(h * HW, HW)]],
            buf.at[pl.ds(h * HW, HW)], gsem).wait()

    def s_issue(jj, buf, ssem):
        pltpu.async_copy(buf, s_sh.at[col_st.at[jj]], ssem, add=True)

    def s_wait(jj, buf, ssem):
        pltpu.make_async_copy(buf, s_sh.at[col_st.at[jj]], ssem).wait()

    def scale_h(jj, buf, h):
        @pl.loop(h * (HW // 16), (h + 1) * (HW // 16))
        def _(grp):
            ew_reg = ew_st[jj, pl.ds(grp * 16, 16)]
            for kk in range(16):
                w = lax.gather(
                    ew_reg, jnp.full((16, 1), kk, jnp.int32),
                    lax.GatherDimensionNumbers(
                        offset_dims=(), collapsed_slice_dims=(0,),
                        start_index_map=(0,)),
                    (1,), mode=lax.GatherScatterMode.PROMISE_IN_BOUNDS)
                e = grp * 16 + kk
                for cc in range(OUT // 16):
                    buf[e, pl.ds(cc * 16, 16)] = (
                        buf[e, pl.ds(cc * 16, 16)] * w)

    g_issue_h(rowof(0), msA, 0, gsA0)
    g_issue_h(rowof(0), msA, 1, gsA1)
    g_issue_h(rowof(1), msB, 0, gsB0)
    g_issue_h(rowof(1), msB, 1, gsB1)

    @pl.loop(0, npair)
    def _(pp):
        w0 = 2 * pp
        w1 = w0 + 1
        pos = pp % (CH // 2)
        nxt = (w0 // CH + 1) * CH
        half = ((nxt // CH) % 2) * CH

        @pl.when(jnp.logical_and(pos == 0, nxt < nwin))
        def _():
            off = base + nxt
            pltpu.async_copy(
                row_hbm.at[pl.ds(off, CH)], row_st.at[pl.ds(half, CH)], stsem)
            pltpu.async_copy(
                col_hbm.at[pl.ds(off, CH)], col_st.at[pl.ds(half, CH)], stsem)
            pltpu.async_copy(
                ew_hbm.at[pl.ds(off, CH)], ew_st.at[pl.ds(half, CH)], stsem)

        jj0 = rowof(w0)
        jj1 = rowof(w1)
        g_wait_h(jj0, msA, 0, gsA0)
        scale_h(jj0, msA, 0)
        g_wait_h(jj0, msA, 1, gsA1)
        scale_h(jj0, msA, 1)
        s_issue(jj0, msA, ssA)
        g_wait_h(jj1, msB, 0, gsB0)
        scale_h(jj1, msB, 0)
        s_wait(jj0, msA, ssA)

        @pl.when(w0 + 2 < nwin)
        def _():
            g_issue_h(rowof(w0 + 2), msA, 0, gsA0)
            g_issue_h(rowof(w0 + 2), msA, 1, gsA1)

        g_wait_h(jj1, msB, 1, gsB1)
        scale_h(jj1, msB, 1)
        s_issue(jj1, msB, ssB)

        @pl.when(jnp.logical_and(pos == CH // 2 - 1, nxt < nwin))
        def _():
            off = base + nxt
            pltpu.make_async_copy(
                row_hbm.at[pl.ds(off, CH)], row_st.at[pl.ds(half, CH)],
                stsem).wait()
            pltpu.make_async_copy(
                col_hbm.at[pl.ds(off, CH)], col_st.at[pl.ds(half, CH)],
                stsem).wait()
            pltpu.make_async_copy(
                ew_hbm.at[pl.ds(off, CH)], ew_st.at[pl.ds(half, CH)],
                stsem).wait()

        s_wait(jj1, msB, ssB)

        @pl.when(w1 + 2 < nwin)
        def _():
            g_issue_h(rowof(w1 + 2), msB, 0, gsB0)
            g_issue_h(rowof(w1 + 2), msB, 1, gsB1)

    plsc.subcore_barrier()
    pltpu.sync_copy(
        s_sh.at[pl.ds(sid * RPS, RPS)],
        out_hbm.at[cid, pl.ds(sid * RPS, RPS)],
    )


def _h_body(xe_ref, wg_ref, h_ref):
    h_ref[...] = lax.dot_general(
        xe_ref[...], wg_ref[...], (((1,), (0,)), ((), ())),
        precision=lax.Precision.HIGHEST,
        preferred_element_type=jnp.float32,
    )


_h_call = pl.pallas_call(
    _h_body,
    grid=(NB,),
    in_specs=[
        pl.BlockSpec((BN_, D + T), lambda i: (i, 0)),
        pl.BlockSpec((D + T, OUT), lambda i: (0, 0)),
    ],
    out_specs=pl.BlockSpec((BN_, OUT), lambda i: (i, 0)),
    out_shape=jax.ShapeDtypeStruct((N, OUT), jnp.float32),
)


def _g_body(degp_ref, h_ref, g_ref):
    deg = 1.0 + degp_ref[0][:, 0:1] + degp_ref[1][:, 0:1]
    dinv = lax.rsqrt(deg)
    g_ref[...] = h_ref[...] * dinv


_g_call = pl.pallas_call(
    _g_body,
    grid=(NB,),
    in_specs=[
        pl.BlockSpec((2, BN_, 16), lambda i: (0, i, 0)),
        pl.BlockSpec((BN_, OUT), lambda i: (i, 0)),
    ],
    out_specs=pl.BlockSpec((BN_, OUT), lambda i: (i, 0)),
    out_shape=jax.ShapeDtypeStruct((N, OUT), jnp.float32),
)


def _agg_body(degp_ref, g_ref, s_ref, agg_ref, st_ref):
    i = pl.program_id(0)
    deg = 1.0 + degp_ref[0][:, 0:1] + degp_ref[1][:, 0:1]
    dinv = lax.rsqrt(deg)
    agg = dinv * (g_ref[...] + s_ref[0] + s_ref[1])
    agg_ref[...] = agg

    @pl.when(i == 0)
    def _():
        st_ref[...] = jnp.zeros_like(st_ref)

    s0 = jnp.sum(agg, axis=0, keepdims=True)
    s1 = jnp.sum(agg * agg, axis=0, keepdims=True)
    st_ref[...] += jnp.concatenate(
        [s0, s1, jnp.zeros((6, OUT), jnp.float32)], axis=0
    )


_agg_call = pl.pallas_call(
    _agg_body,
    grid=(NB,),
    in_specs=[
        pl.BlockSpec((2, BN_, 16), lambda i: (0, i, 0)),
        pl.BlockSpec((BN_, OUT), lambda i: (i, 0)),
        pl.BlockSpec((2, BN_, OUT), lambda i: (0, i, 0)),
    ],
    out_specs=[
        pl.BlockSpec((BN_, OUT), lambda i: (i, 0)),
        pl.BlockSpec((8, OUT), lambda i: (0, 0)),
    ],
    out_shape=[
        jax.ShapeDtypeStruct((N, OUT), jnp.float32),
        jax.ShapeDtypeStruct((8, OUT), jnp.float32),
    ],
)


def _out_body(agg_ref, st_ref, gam_ref, bet_ref, wl_ref, bl_ref, o_ref):
    inv_n = 1.0 / N
    mean = st_ref[0:1, :] * inv_n
    ex2 = st_ref[1:2, :] * inv_n
    var = ex2 - mean * mean
    rstd = lax.rsqrt(var + 1e-5)
    scale = gam_ref[...] * rstd
    shift = bet_ref[...] - mean * scale
    bn = agg_ref[...] * scale + shift
    act = jnp.where(bn >= 0.0, bn, 0.01 * bn)
    o_ref[...] = lax.dot_general(
        act, wl_ref[...], (((1,), (1,)), ((), ())),
        precision=lax.Precision.HIGHEST,
        preferred_element_type=jnp.float32,
    ) + bl_ref[...]


_out_call = pl.pallas_call(
    _out_body,
    grid=(NB,),
    in_specs=[
        pl.BlockSpec((BN_, OUT), lambda i: (i, 0)),
        pl.BlockSpec((8, OUT), lambda i: (0, 0)),
        pl.BlockSpec((1, OUT), lambda i: (0, 0)),
        pl.BlockSpec((1, OUT), lambda i: (0, 0)),
        pl.BlockSpec((OUT, OUT), lambda i: (0, 0)),
        pl.BlockSpec((1, OUT), lambda i: (0, 0)),
    ],
    out_specs=pl.BlockSpec((BN_, OUT), lambda i: (i, 0)),
    out_shape=jax.ShapeDtypeStruct((N, OUT), jnp.float32),
)


def kernel(x, edge_index, edge_weight, time_diff, time_segment,
           morning_freq, evening_freq, nighttime_freq, other_freq,
           W_gcn, b_gcn, bn_gamma, bn_beta, W_lin, b_lin):
    del time_segment, morning_freq, evening_freq, nighttime_freq, b_gcn

    row = edge_index[0]
    col = edge_index[1]
    pad = EP - E
    zi = jnp.zeros((pad,), jnp.int32)
    rowp = jnp.concatenate([row, zi]).reshape(EROWS, WIN)
    colp = jnp.concatenate([col, zi]).reshape(EROWS, WIN)
    ewp = jnp.concatenate(
        [edge_weight, jnp.zeros((pad,), jnp.float32)]
    ).reshape(EROWS, WIN)

    enc = time_diff * other_freq * (math.pi / 24.0)
    xe = jnp.concatenate(
        [x, jnp.broadcast_to(enc[None, :], (N, T))], axis=1
    )

    degp = _deg_kernel(colp, ewp)
    h = _h_call(xe, W_gcn)
    g = _g_call(degp, h)
    s = _prop_kernel(g, rowp, colp, ewp)
    agg, st = _agg_call(degp, g, s)
    out = _out_call(
        agg, st,
        bn_gamma.reshape(1, OUT), bn_beta.reshape(1, OUT),
        W_lin, b_lin.reshape(1, OUT),
    )
    return out


# speedup vs baseline: 1.0510x; 1.0510x over previous
import dataclasses
import functools
import math

import jax
import jax.numpy as jnp
from jax import lax
from jax.experimental import pallas as pl
from jax.experimental.pallas import tpu as pltpu
from jax.experimental.pallas import tpu_sc as plsc

N = 10000
E = 320000
D = 128
T = 16
OUT = 128

NC = 2
NS = 16
WIN = 128
TPW = 80
CH = 8
T0 = 136
T1 = 24
NW0 = NS * T0
EP = NC * NS * TPW * WIN
EROWS = EP // WIN
NP = 10240
RPT = NP // NS
NPS = 10112
RPS = NPS // NS

NB = 25
BN_ = N // NB

_mesh = plsc.VectorSubcoreMesh(core_axis_name="c", subcore_axis_name="s")

_sc_params = pltpu.CompilerParams()
if "needs_layout_passes" in pltpu.CompilerParams.__dataclass_fields__:
    _sc_params = dataclasses.replace(_sc_params, needs_layout_passes=False)


@functools.partial(
    pl.kernel,
    out_type=jax.ShapeDtypeStruct((NC, NP, 16), jnp.float32),
    mesh=_mesh,
    scratch_types=[
        pltpu.VMEM_SHARED((NP, 16), jnp.float32),
        pltpu.VMEM((TPW, WIN), jnp.int32),
        pltpu.VMEM((TPW, WIN), jnp.float32),
        pltpu.VMEM((WIN, 16), jnp.float32),
        pltpu.VMEM((RPT, 16), jnp.float32),
    ],
    compiler_params=_sc_params,
)
def _deg_kernel(col_hbm, ew_hbm, out_hbm, deg_sh, col_v, ew_v, buf, zbuf):
    cid = lax.axis_index("c")
    sid = lax.axis_index("s")
    tile = cid * NS + sid
    zero16 = jnp.zeros((16,), jnp.float32)

    @pl.loop(0, RPT)
    def _(r):
        zbuf[r, :] = zero16

    pltpu.sync_copy(zbuf, deg_sh.at[pl.ds(sid * RPT, RPT)])

    @pl.loop(0, WIN)
    def _(r):
        buf[r, :] = zero16

    base = tile * TPW
    pltpu.sync_copy(col_hbm.at[pl.ds(base, TPW)], col_v)
    pltpu.sync_copy(ew_hbm.at[pl.ds(base, TPW)], ew_v)

    iota16 = lax.iota(jnp.int32, 16)
    zi16 = jnp.zeros((16,), jnp.int32)
    plsc.subcore_barrier()

    @pl.loop(0, TPW)
    def _(j):
        @pl.loop(0, WIN // 16)
        def _(grp):
            ew_reg = ew_v[j, pl.ds(grp * 16, 16)]
            plsc.store_scatter(buf, [iota16 + grp * 16, zi16], ew_reg)

        pltpu.sync_copy(buf, deg_sh.at[col_v.at[j]], add=True)

    plsc.subcore_barrier()
    pltpu.sync_copy(
        deg_sh.at[pl.ds(sid * RPT, RPT)],
        out_hbm.at[cid, pl.ds(sid * RPT, RPT)],
    )


@functools.partial(
    pl.kernel,
    out_type=jax.ShapeDtypeStruct((NC, NPS, OUT), jnp.float32),
    mesh=_mesh,
    scratch_types=[
        pltpu.VMEM_SHARED((NPS, OUT), jnp.float32),
        pltpu.VMEM((2 * CH, WIN), jnp.int32),
        pltpu.VMEM((2 * CH, WIN), jnp.int32),
        pltpu.VMEM((2 * CH, WIN), jnp.float32),
        pltpu.VMEM((WIN, OUT), jnp.float32),
        pltpu.VMEM((WIN, OUT), jnp.float32),
        pltpu.SemaphoreType.DMA,
        pltpu.SemaphoreType.DMA,
        pltpu.SemaphoreType.DMA,
        pltpu.SemaphoreType.DMA,
        pltpu.SemaphoreType.DMA,
    ],
    compiler_params=_sc_params,
)
def _prop_kernel(g_hbm, row_hbm, col_hbm, ew_hbm, out_hbm,
                 s_sh, row_st, col_st, ew_st, msA, msB,
                 gsA, gsB, ssA, ssB, stsem):
    cid = lax.axis_index("c")
    sid = lax.axis_index("s")
    zero16 = jnp.zeros((16,), jnp.float32)

    base = jnp.where(cid == 0, sid * T0, NW0 + sid * T1)
    nwin = jnp.where(cid == 0, T0, T1)
    npair = nwin // 2

    @pl.loop(0, WIN)
    def _(r):
        @pl.loop(0, OUT // 16)
        def _(cc):
            msA[r, pl.ds(cc * 16, 16)] = zero16

    @pl.loop(0, RPS // WIN)
    def _(m):
        pltpu.sync_copy(msA, s_sh.at[pl.ds(sid * RPS + m * WIN, WIN)])
    _TAIL = RPS - (RPS // WIN) * WIN
    if _TAIL:
        pltpu.sync_copy(
            msA.at[pl.ds(0, _TAIL)],
            s_sh.at[pl.ds(sid * RPS + (RPS // WIN) * WIN, _TAIL)])

    pltpu.sync_copy(row_hbm.at[pl.ds(base, CH)], row_st.at[pl.ds(0, CH)])
    pltpu.sync_copy(col_hbm.at[pl.ds(base, CH)], col_st.at[pl.ds(0, CH)])
    pltpu.sync_copy(ew_hbm.at[pl.ds(base, CH)], ew_st.at[pl.ds(0, CH)])

    plsc.subcore_barrier()

    def rowof(w):
        return ((w // CH) % 2) * CH + w % CH

    def g_issue(jj, buf, gsem):
        pltpu.async_copy(g_hbm.at[row_st.at[jj]], buf, gsem)

    def g_wait(jj, buf, gsem):
        pltpu.make_async_copy(g_hbm.at[row_st.at[jj]], buf, gsem).wait()

    def s_issue(jj, buf, ssem):
        pltpu.async_copy(buf, s_sh.at[col_st.at[jj]], ssem, add=True)

    def s_wait(jj, buf, ssem):
        pltpu.make_async_copy(buf, s_sh.at[col_st.at[jj]], ssem).wait()

    def scale(jj, buf):
        @pl.loop(0, WIN // 16)
        def _(grp):
            ew_reg = ew_st[jj, pl.ds(grp * 16, 16)]
            for kk in range(16):
                w = lax.gather(
                    ew_reg, jnp.full((16, 1), kk, jnp.int32),
                    lax.GatherDimensionNumbers(
                        offset_dims=(), collapsed_slice_dims=(0,),
                        start_index_map=(0,)),
                    (1,), mode=lax.GatherScatterMode.PROMISE_IN_BOUNDS)
                e = grp * 16 + kk
                for cc in range(OUT // 16):
                    buf[e, pl.ds(cc * 16, 16)] = (
                        buf[e, pl.ds(cc * 16, 16)] * w)

    g_issue(rowof(0), msA, gsA)
    g_issue(rowof(1), msB, gsB)

    @pl.loop(0, npair)
    def _(pp):
        w0 = 2 * pp
        w1 = w0 + 1
        pos = pp % (CH // 2)
        nxt = (w0 // CH + 1) * CH
        half = ((nxt // CH) % 2) * CH

        @pl.when(jnp.logical_and(pos == 0, nxt < nwin))
        def _():
            off = base + nxt
            pltpu.async_copy(
                row_hbm.at[pl.ds(off, CH)], row_st.at[pl.ds(half, CH)], stsem)
            pltpu.async_copy(
                col_hbm.at[pl.ds(off, CH)], col_st.at[pl.ds(half, CH)], stsem)
            pltpu.async_copy(
                ew_hbm.at[pl.ds(off, CH)], ew_st.at[pl.ds(half, CH)], stsem)

        jj0 = rowof(w0)
        jj1 = rowof(w1)
        g_wait(jj0, msA, gsA)
        scale(jj0, msA)
        s_issue(jj0, msA, ssA)
        g_wait(jj1, msB, gsB)
        s_wait(jj0, msA, ssA)

        @pl.when(w0 + 2 < nwin)
        def _():
            g_issue(rowof(w0 + 2), msA, gsA)

        scale(jj1, msB)
        s_issue(jj1, msB, ssB)

        @pl.when(jnp.logical_and(pos == CH // 2 - 1, nxt < nwin))
        def _():
            off = base + nxt
            pltpu.make_async_copy(
                row_hbm.at[pl.ds(off, CH)], row_st.at[pl.ds(half, CH)],
                stsem).wait()
            pltpu.make_async_copy(
                col_hbm.at[pl.ds(off, CH)], col_st.at[pl.ds(half, CH)],
                stsem).wait()
            pltpu.make_async_copy(
                ew_hbm.at[pl.ds(off, CH)], ew_st.at[pl.ds(half, CH)],
                stsem).wait()

        s_wait(jj1, msB, ssB)

        @pl.when(w1 + 2 < nwin)
        def _():
            g_issue(rowof(w1 + 2), msB, gsB)

    plsc.subcore_barrier()
    pltpu.sync_copy(
        s_sh.at[pl.ds(sid * RPS, RPS)],
        out_hbm.at[cid, pl.ds(sid * RPS, RPS)],
    )


def _h_body(xe_ref, wg_ref, h_ref):
    h_ref[...] = lax.dot_general(
        xe_ref[...], wg_ref[...], (((1,), (0,)), ((), ())),
        precision=lax.Precision.HIGHEST,
        preferred_element_type=jnp.float32,
    )


_h_call = pl.pallas_call(
    _h_body,
    grid=(NB,),
    in_specs=[
        pl.BlockSpec((BN_, D + T), lambda i: (i, 0)),
        pl.BlockSpec((D + T, OUT), lambda i: (0, 0)),
    ],
    out_specs=pl.BlockSpec((BN_, OUT), lambda i: (i, 0)),
    out_shape=jax.ShapeDtypeStruct((N, OUT), jnp.float32),
)


def _g_body(degp_ref, h_ref, g_ref):
    deg = 1.0 + degp_ref[0][:, 0:1] + degp_ref[1][:, 0:1]
    dinv = lax.rsqrt(deg)
    g_ref[...] = h_ref[...] * dinv


_g_call = pl.pallas_call(
    _g_body,
    grid=(NB,),
    in_specs=[
        pl.BlockSpec((2, BN_, 16), lambda i: (0, i, 0)),
        pl.BlockSpec((BN_, OUT), lambda i: (i, 0)),
    ],
    out_specs=pl.BlockSpec((BN_, OUT), lambda i: (i, 0)),
    out_shape=jax.ShapeDtypeStruct((N, OUT), jnp.float32),
)


def _agg_body(degp_ref, g_ref, s_ref, agg_ref, st_ref):
    i = pl.program_id(0)
    deg = 1.0 + degp_ref[0][:, 0:1] + degp_ref[1][:, 0:1]
    dinv = lax.rsqrt(deg)
    agg = dinv * (g_ref[...] + s_ref[0] + s_ref[1])
    agg_ref[...] = agg

    @pl.when(i == 0)
    def _():
        st_ref[...] = jnp.zeros_like(st_ref)

    s0 = jnp.sum(agg, axis=0, keepdims=True)
    s1 = jnp.sum(agg * agg, axis=0, keepdims=True)
    st_ref[...] += jnp.concatenate(
        [s0, s1, jnp.zeros((6, OUT), jnp.float32)], axis=0
    )


_agg_call = pl.pallas_call(
    _agg_body,
    grid=(NB,),
    in_specs=[
        pl.BlockSpec((2, BN_, 16), lambda i: (0, i, 0)),
        pl.BlockSpec((BN_, OUT), lambda i: (i, 0)),
        pl.BlockSpec((2, BN_, OUT), lambda i: (0, i, 0)),
    ],
    out_specs=[
        pl.BlockSpec((BN_, OUT), lambda i: (i, 0)),
        pl.BlockSpec((8, OUT), lambda i: (0, 0)),
    ],
    out_shape=[
        jax.ShapeDtypeStruct((N, OUT), jnp.float32),
        jax.ShapeDtypeStruct((8, OUT), jnp.float32),
    ],
)


def _out_body(agg_ref, st_ref, gam_ref, bet_ref, wl_ref, bl_ref, o_ref):
    inv_n = 1.0 / N
    mean = st_ref[0:1, :] * inv_n
    ex2 = st_ref[1:2, :] * inv_n
    var = ex2 - mean * mean
    rstd = lax.rsqrt(var + 1e-5)
    scale = gam_ref[...] * rstd
    shift = bet_ref[...] - mean * scale
    bn = agg_ref[...] * scale + shift
    act = jnp.where(bn >= 0.0, bn, 0.01 * bn)
    o_ref[...] = lax.dot_general(
        act, wl_ref[...], (((1,), (1,)), ((), ())),
        precision=lax.Precision.HIGHEST,
        preferred_element_type=jnp.float32,
    ) + bl_ref[...]


_out_call = pl.pallas_call(
    _out_body,
    grid=(NB,),
    in_specs=[
        pl.BlockSpec((BN_, OUT), lambda i: (i, 0)),
        pl.BlockSpec((8, OUT), lambda i: (0, 0)),
        pl.BlockSpec((1, OUT), lambda i: (0, 0)),
        pl.BlockSpec((1, OUT), lambda i: (0, 0)),
        pl.BlockSpec((OUT, OUT), lambda i: (0, 0)),
        pl.BlockSpec((1, OUT), lambda i: (0, 0)),
    ],
    out_specs=pl.BlockSpec((BN_, OUT), lambda i: (i, 0)),
    out_shape=jax.ShapeDtypeStruct((N, OUT), jnp.float32),
)


def kernel(x, edge_index, edge_weight, time_diff, time_segment,
           morning_freq, evening_freq, nighttime_freq, other_freq,
           W_gcn, b_gcn, bn_gamma, bn_beta, W_lin, b_lin):
    del time_segment, morning_freq, evening_freq, nighttime_freq, b_gcn

    row = edge_index[0]
    col = edge_index[1]
    pad = EP - E
    zi = jnp.zeros((pad,), jnp.int32)
    rowp = jnp.concatenate([row, zi]).reshape(EROWS, WIN)
    colp = jnp.concatenate([col, zi]).reshape(EROWS, WIN)
    ewp = jnp.concatenate(
        [edge_weight, jnp.zeros((pad,), jnp.float32)]
    ).reshape(EROWS, WIN)

    enc = time_diff * other_freq * (math.pi / 24.0)
    xe = jnp.concatenate(
        [x, jnp.broadcast_to(enc[None, :], (N, T))], axis=1
    )

    degp = _deg_kernel(colp, ewp)
    h = _h_call(xe, W_gcn)
    g = _g_call(degp, h)
    s = _prop_kernel(g, rowp, colp, ewp)
    agg, st = _agg_call(degp, g, s)
    out = _out_call(
        agg, st,
        bn_gamma.reshape(1, OUT), bn_beta.reshape(1, OUT),
        W_lin, b_lin.reshape(1, OUT),
    )
    return out


# speedup vs baseline: 1.0571x; 1.0058x over previous
import dataclasses
import functools
import math

import jax
import jax.numpy as jnp
from jax import lax
from jax.experimental import pallas as pl
from jax.experimental.pallas import tpu as pltpu
from jax.experimental.pallas import tpu_sc as plsc

N = 10000
E = 320000
D = 128
T = 16
OUT = 128

NC = 2
NS = 16
WIN = 128
TPW = 80
CH = 8
T0 = 136
T1 = 24
NW0 = NS * T0
EP = NC * NS * TPW * WIN
EROWS = EP // WIN
NP = 10240
RPT = NP // NS
NPS = 10112
RPS = NPS // NS

NB = 25
BN_ = N // NB

_mesh = plsc.VectorSubcoreMesh(core_axis_name="c", subcore_axis_name="s")

_sc_params = pltpu.CompilerParams()
if "needs_layout_passes" in pltpu.CompilerParams.__dataclass_fields__:
    _sc_params = dataclasses.replace(_sc_params, needs_layout_passes=False)


@functools.partial(
    pl.kernel,
    out_type=jax.ShapeDtypeStruct((NC, NP, 16), jnp.float32),
    mesh=_mesh,
    scratch_types=[
        pltpu.VMEM_SHARED((NP, 16), jnp.float32),
        pltpu.VMEM((TPW, WIN), jnp.int32),
        pltpu.VMEM((TPW, WIN), jnp.float32),
        pltpu.VMEM((WIN, 16), jnp.float32),
        pltpu.VMEM((RPT, 16), jnp.float32),
    ],
    compiler_params=_sc_params,
)
def _deg_kernel(col_hbm, ew_hbm, out_hbm, deg_sh, col_v, ew_v, buf, zbuf):
    cid = lax.axis_index("c")
    sid = lax.axis_index("s")
    tile = cid * NS + sid
    zero16 = jnp.zeros((16,), jnp.float32)

    @pl.loop(0, RPT)
    def _(r):
        zbuf[r, :] = zero16

    pltpu.sync_copy(zbuf, deg_sh.at[pl.ds(sid * RPT, RPT)])

    @pl.loop(0, WIN)
    def _(r):
        buf[r, :] = zero16

    base = tile * TPW
    pltpu.sync_copy(col_hbm.at[pl.ds(base, TPW)], col_v)
    pltpu.sync_copy(ew_hbm.at[pl.ds(base, TPW)], ew_v)

    iota16 = lax.iota(jnp.int32, 16)
    zi16 = jnp.zeros((16,), jnp.int32)
    plsc.subcore_barrier()

    @pl.loop(0, TPW)
    def _(j):
        @pl.loop(0, WIN // 16)
        def _(grp):
            ew_reg = ew_v[j, pl.ds(grp * 16, 16)]
            plsc.store_scatter(buf, [iota16 + grp * 16, zi16], ew_reg)

        pltpu.sync_copy(buf, deg_sh.at[col_v.at[j]], add=True)

    plsc.subcore_barrier()
    pltpu.sync_copy(
        deg_sh.at[pl.ds(sid * RPT, RPT)],
        out_hbm.at[cid, pl.ds(sid * RPT, RPT)],
    )


@functools.partial(
    pl.kernel,
    out_type=jax.ShapeDtypeStruct((NC, NPS, OUT), jnp.float32),
    mesh=_mesh,
    scratch_types=[
        pltpu.VMEM_SHARED((NPS, OUT), jnp.float32),
        pltpu.VMEM((2 * CH, WIN), jnp.int32),
        pltpu.VMEM((2 * CH, WIN), jnp.int32),
        pltpu.VMEM((2 * CH, WIN), jnp.float32),
        pltpu.VMEM((WIN, OUT), jnp.float32),
        pltpu.VMEM((WIN, OUT), jnp.float32),
        pltpu.SemaphoreType.DMA,
        pltpu.SemaphoreType.DMA,
        pltpu.SemaphoreType.DMA,
        pltpu.SemaphoreType.DMA,
        pltpu.SemaphoreType.DMA,
    ],
    compiler_params=_sc_params,
)
def _prop_kernel(g_hbm, row_hbm, col_hbm, ew_hbm, out_hbm,
                 s_sh, row_st, col_st, ew_st, msA, msB,
                 gsA, gsB, ssA, ssB, stsem):
    cid = lax.axis_index("c")
    sid = lax.axis_index("s")
    zero16 = jnp.zeros((16,), jnp.float32)

    base = jnp.where(cid == 0, sid * T0, NW0 + sid * T1)
    nwin = jnp.where(cid == 0, T0, T1)
    npair = nwin // 2

    @pl.loop(0, WIN)
    def _(r):
        @pl.loop(0, OUT // 16)
        def _(cc):
            msA[r, pl.ds(cc * 16, 16)] = zero16

    @pl.loop(0, RPS // WIN)
    def _(m):
        pltpu.sync_copy(msA, s_sh.at[pl.ds(sid * RPS + m * WIN, WIN)])
    _TAIL = RPS - (RPS // WIN) * WIN
    if _TAIL:
        pltpu.sync_copy(
            msA.at[pl.ds(0, _TAIL)],
            s_sh.at[pl.ds(sid * RPS + (RPS // WIN) * WIN, _TAIL)])

    pltpu.sync_copy(row_hbm.at[pl.ds(base, CH)], row_st.at[pl.ds(0, CH)])
    pltpu.sync_copy(col_hbm.at[pl.ds(base, CH)], col_st.at[pl.ds(0, CH)])
    pltpu.sync_copy(ew_hbm.at[pl.ds(base, CH)], ew_st.at[pl.ds(0, CH)])

    plsc.subcore_barrier()

    def rowof(w):
        return ((w // CH) % 2) * CH + w % CH

    def g_issue(jj, buf, gsem):
        pltpu.async_copy(g_hbm.at[row_st.at[jj]], buf, gsem)

    def g_wait(jj, buf, gsem):
        pltpu.make_async_copy(g_hbm.at[row_st.at[jj]], buf, gsem).wait()

    def s_issue(jj, buf, ssem):
        pltpu.async_copy(buf, s_sh.at[col_st.at[jj]], ssem, add=True)

    def s_wait(jj, buf, ssem):
        pltpu.make_async_copy(buf, s_sh.at[col_st.at[jj]], ssem).wait()

    def scale(jj, buf):
        @pl.loop(0, WIN // 16)
        def _(grp):
            ew_reg = ew_st[jj, pl.ds(grp * 16, 16)]
            for kk in range(16):
                w = lax.gather(
                    ew_reg, jnp.full((16, 1), kk, jnp.int32),
                    lax.GatherDimensionNumbers(
                        offset_dims=(), collapsed_slice_dims=(0,),
                        start_index_map=(0,)),
                    (1,), mode=lax.GatherScatterMode.PROMISE_IN_BOUNDS)
                e = grp * 16 + kk
                for cc in range(OUT // 16):
                    buf[e, pl.ds(cc * 16, 16)] = (
                        buf[e, pl.ds(cc * 16, 16)] * w)

    g_issue(rowof(0), msA, gsA)
    g_issue(rowof(1), msB, gsB)

    @pl.loop(0, npair)
    def _(pp):
        w0 = 2 * pp
        w1 = w0 + 1
        pos = pp % (CH // 2)
        nxt = (w0 // CH + 1) * CH
        half = ((nxt // CH) % 2) * CH

        @pl.when(jnp.logical_and(pos == 0, nxt < nwin))
        def _():
            off = base + nxt
            pltpu.async_copy(
                row_hbm.at[pl.ds(off, CH)], row_st.at[pl.ds(half, CH)], stsem)
            pltpu.async_copy(
                col_hbm.at[pl.ds(off, CH)], col_st.at[pl.ds(half, CH)], stsem)
            pltpu.async_copy(
                ew_hbm.at[pl.ds(off, CH)], ew_st.at[pl.ds(half, CH)], stsem)

        jj0 = rowof(w0)
        jj1 = rowof(w1)
        g_wait(jj0, msA, gsA)
        scale(jj0, msA)
        s_issue(jj0, msA, ssA)
        g_wait(jj1, msB, gsB)
        scale(jj1, msB)
        s_issue(jj1, msB, ssB)

        @pl.when(jnp.logical_and(pos == CH // 2 - 1, nxt < nwin))
        def _():
            off = base + nxt
            pltpu.make_async_copy(
                row_hbm.at[pl.ds(off, CH)], row_st.at[pl.ds(half, CH)],
                stsem).wait()
            pltpu.make_async_copy(
                col_hbm.at[pl.ds(off, CH)], col_st.at[pl.ds(half, CH)],
                stsem).wait()
            pltpu.make_async_copy(
                ew_hbm.at[pl.ds(off, CH)], ew_st.at[pl.ds(half, CH)],
                stsem).wait()

        s_wait(jj0, msA, ssA)

        @pl.when(w0 + 2 < nwin)
        def _():
            g_issue(rowof(w0 + 2), msA, gsA)

        s_wait(jj1, msB, ssB)

        @pl.when(w1 + 2 < nwin)
        def _():
            g_issue(rowof(w1 + 2), msB, gsB)

    plsc.subcore_barrier()
    pltpu.sync_copy(
        s_sh.at[pl.ds(sid * RPS, RPS)],
        out_hbm.at[cid, pl.ds(sid * RPS, RPS)],
    )


def _h_body(xe_ref, wg_ref, h_ref):
    h_ref[...] = lax.dot_general(
        xe_ref[...], wg_ref[...], (((1,), (0,)), ((), ())),
        precision=lax.Precision.HIGHEST,
        preferred_element_type=jnp.float32,
    )


_h_call = pl.pallas_call(
    _h_body,
    grid=(NB,),
    in_specs=[
        pl.BlockSpec((BN_, D + T), lambda i: (i, 0)),
        pl.BlockSpec((D + T, OUT), lambda i: (0, 0)),
    ],
    out_specs=pl.BlockSpec((BN_, OUT), lambda i: (i, 0)),
    out_shape=jax.ShapeDtypeStruct((N, OUT), jnp.float32),
)


def _g_body(degp_ref, h_ref, g_ref):
    deg = 1.0 + degp_ref[0][:, 0:1] + degp_ref[1][:, 0:1]
    dinv = lax.rsqrt(deg)
    g_ref[...] = h_ref[...] * dinv


_g_call = pl.pallas_call(
    _g_body,
    grid=(NB,),
    in_specs=[
        pl.BlockSpec((2, BN_, 16), lambda i: (0, i, 0)),
        pl.BlockSpec((BN_, OUT), lambda i: (i, 0)),
    ],
    out_specs=pl.BlockSpec((BN_, OUT), lambda i: (i, 0)),
    out_shape=jax.ShapeDtypeStruct((N, OUT), jnp.float32),
)


def _agg_body(degp_ref, g_ref, s_ref, agg_ref, st_ref):
    i = pl.program_id(0)
    deg = 1.0 + degp_ref[0][:, 0:1] + degp_ref[1][:, 0:1]
    dinv = lax.rsqrt(deg)
    agg = dinv * (g_ref[...] + s_ref[0] + s_ref[1])
    agg_ref[...] = agg

    @pl.when(i == 0)
    def _():
        st_ref[...] = jnp.zeros_like(st_ref)

    s0 = jnp.sum(agg, axis=0, keepdims=True)
    s1 = jnp.sum(agg * agg, axis=0, keepdims=True)
    st_ref[...] += jnp.concatenate(
        [s0, s1, jnp.zeros((6, OUT), jnp.float32)], axis=0
    )


_agg_call = pl.pallas_call(
    _agg_body,
    grid=(NB,),
    in_specs=[
        pl.BlockSpec((2, BN_, 16), lambda i: (0, i, 0)),
        pl.BlockSpec((BN_, OUT), lambda i: (i, 0)),
        pl.BlockSpec((2, BN_, OUT), lambda i: (0, i, 0)),
    ],
    out_specs=[
        pl.BlockSpec((BN_, OUT), lambda i: (i, 0)),
        pl.BlockSpec((8, OUT), lambda i: (0, 0)),
    ],
    out_shape=[
        jax.ShapeDtypeStruct((N, OUT), jnp.float32),
        jax.ShapeDtypeStruct((8, OUT), jnp.float32),
    ],
)


def _out_body(agg_ref, st_ref, gam_ref, bet_ref, wl_ref, bl_ref, o_ref):
    inv_n = 1.0 / N
    mean = st_ref[0:1, :] * inv_n
    ex2 = st_ref[1:2, :] * inv_n
    var = ex2 - mean * mean
    rstd = lax.rsqrt(var + 1e-5)
    scale = gam_ref[...] * rstd
    shift = bet_ref[...] - mean * scale
    bn = agg_ref[...] * scale + shift
    act = jnp.where(bn >= 0.0, bn, 0.01 * bn)
    o_ref[...] = lax.dot_general(
        act, wl_ref[...], (((1,), (1,)), ((), ())),
        precision=lax.Precision.HIGHEST,
        preferred_element_type=jnp.float32,
    ) + bl_ref[...]


_out_call = pl.pallas_call(
    _out_body,
    grid=(NB,),
    in_specs=[
        pl.BlockSpec((BN_, OUT), lambda i: (i, 0)),
        pl.BlockSpec((8, OUT), lambda i: (0, 0)),
        pl.BlockSpec((1, OUT), lambda i: (0, 0)),
        pl.BlockSpec((1, OUT), lambda i: (0, 0)),
        pl.BlockSpec((OUT, OUT), lambda i: (0, 0)),
        pl.BlockSpec((1, OUT), lambda i: (0, 0)),
    ],
    out_specs=pl.BlockSpec((BN_, OUT), lambda i: (i, 0)),
    out_shape=jax.ShapeDtypeStruct((N, OUT), jnp.float32),
)


def kernel(x, edge_index, edge_weight, time_diff, time_segment,
           morning_freq, evening_freq, nighttime_freq, other_freq,
           W_gcn, b_gcn, bn_gamma, bn_beta, W_lin, b_lin):
    del time_segment, morning_freq, evening_freq, nighttime_freq, b_gcn

    row = edge_index[0]
    col = edge_index[1]
    pad = EP - E
    zi = jnp.zeros((pad,), jnp.int32)
    rowp = jnp.concatenate([row, zi]).reshape(EROWS, WIN)
    colp = jnp.concatenate([col, zi]).reshape(EROWS, WIN)
    ewp = jnp.concatenate(
        [edge_weight, jnp.zeros((pad,), jnp.float32)]
    ).reshape(EROWS, WIN)

    enc = time_diff * other_freq * (math.pi / 24.0)
    xe = jnp.concatenate(
        [x, jnp.broadcast_to(enc[None, :], (N, T))], axis=1
    )

    degp = _deg_kernel(colp, ewp)
    h = _h_call(xe, W_gcn)
    g = _g_call(degp, h)
    s = _prop_kernel(g, rowp, colp, ewp)
    agg, st = _agg_call(degp, g, s)
    out = _out_call(
        agg, st,
        bn_gamma.reshape(1, OUT), bn_beta.reshape(1, OUT),
        W_lin, b_lin.reshape(1, OUT),
    )
    return out
